# baseline (speedup 1.0000x reference)
import jax
import jax.numpy as jnp
from jax import lax
from jax.experimental import pallas as pl
from jax.experimental.pallas import tpu as pltpu
from jax.experimental.pallas import tpu_sc as plsc

_B = 65536
_L = 16
_NC = 2
_NS = 16
_NW = _NC * _NS
_CHUNK = _B // _NW


def _body(op_hbm, a_hbm, b_hbm, c_hbm, out_hbm,
          op_v, a_v, b_v, c_v, o_v, sem):
    wid = lax.axis_index("s") * _NC + lax.axis_index("c")
    base = wid * _CHUNK
    sl = pl.ds(base, _CHUNK)
    cps = [pltpu.async_copy(src, dst, sem)
           for src, dst in ((op_hbm.at[sl], op_v), (a_hbm.at[sl], a_v),
                            (b_hbm.at[sl], b_v), (c_hbm.at[sl], c_v))]
    for cp in cps:
        cp.wait()

    @plsc.parallel_loop(0, _CHUNK, _L, unroll=2)
    def step(j):
        off = pl.multiple_of(j, _L)
        op = op_v[pl.ds(off, _L)]
        a = a_v[pl.ds(off, _L)]
        b = b_v[pl.ds(off, _L)]
        c = c_v[pl.ds(off, _L)]
        r_add = (a + b + c) & 255
        r_and = a & b
        r_or = a | b
        r_xor = a ^ b
        r_asl = (a << 1) & 255
        r_lsr = a >> 1
        r_inc = (a + 1) & 255
        r_dec = (a - 1) & 255
        res = jnp.where(
            op < 4,
            jnp.where(op < 2,
                      jnp.where(op == 0, r_add, r_and),
                      jnp.where(op == 2, r_or, r_xor)),
            jnp.where(op < 6,
                      jnp.where(op == 4, r_asl, r_lsr),
                      jnp.where(op == 6, r_inc, r_dec)))
        o_v[pl.ds(off, _L)] = res

    pltpu.sync_copy(o_v, out_hbm.at[sl])


def kernel(opcode, a, b, carry, emb_table, signatures, atom_positions,
           composition_table):
    del emb_table, signatures, atom_positions, composition_table
    mesh = plsc.VectorSubcoreMesh(core_axis_name="c", subcore_axis_name="s")
    f = pl.kernel(
        _body,
        mesh=mesh,
        out_type=jax.ShapeDtypeStruct((_B,), jnp.int32),
        scratch_types=[pltpu.VMEM((_CHUNK,), jnp.int32) for _ in range(5)]
        + [pltpu.SemaphoreType.DMA],
    )
    return f(opcode, a, b, carry)


# speedup vs baseline: 1.0266x; 1.0266x over previous
import jax
import jax.numpy as jnp
from jax import lax
from jax.experimental import pallas as pl
from jax.experimental.pallas import tpu as pltpu
from jax.experimental.pallas import tpu_sc as plsc

_B = 65536
_L = 16
_NC = 1
_NS = 16
_NW = _NC * _NS
_CHUNK = _B // _NW


def _body(op_hbm, a_hbm, b_hbm, c_hbm, out_hbm,
          op_v, a_v, b_v, c_v, o_v, sem):
    wid = lax.axis_index("s") * _NC + lax.axis_index("c")
    base = wid * _CHUNK
    sl = pl.ds(base, _CHUNK)
    cps = [pltpu.async_copy(src, dst, sem)
           for src, dst in ((op_hbm.at[sl], op_v), (a_hbm.at[sl], a_v),
                            (b_hbm.at[sl], b_v), (c_hbm.at[sl], c_v))]
    for cp in cps:
        cp.wait()

    @plsc.parallel_loop(0, _CHUNK, _L, unroll=4)
    def step(j):
        off = pl.multiple_of(j, _L)
        op = op_v[pl.ds(off, _L)]
        a = a_v[pl.ds(off, _L)]
        b = b_v[pl.ds(off, _L)]
        c = c_v[pl.ds(off, _L)]
        r_add = (a + b + c) & 255
        r_and = a & b
        r_or = a | b
        r_xor = a ^ b
        r_asl = (a << 1) & 255
        r_lsr = a >> 1
        r_inc = (a + 1) & 255
        r_dec = (a - 1) & 255
        res = jnp.where(
            op < 4,
            jnp.where(op < 2,
                      jnp.where(op == 0, r_add, r_and),
                      jnp.where(op == 2, r_or, r_xor)),
            jnp.where(op < 6,
                      jnp.where(op == 4, r_asl, r_lsr),
                      jnp.where(op == 6, r_inc, r_dec)))
        o_v[pl.ds(off, _L)] = res

    pltpu.sync_copy(o_v, out_hbm.at[sl])


def kernel(opcode, a, b, carry, emb_table, signatures, atom_positions,
           composition_table):
    del emb_table, signatures, atom_positions, composition_table
    mesh = plsc.VectorSubcoreMesh(core_axis_name="c", subcore_axis_name="s", num_cores=1)
    f = pl.kernel(
        _body,
        mesh=mesh,
        out_type=jax.ShapeDtypeStruct((_B,), jnp.int32),
        scratch_types=[pltpu.VMEM((_CHUNK,), jnp.int32) for _ in range(5)]
        + [pltpu.SemaphoreType.DMA],
    )
    return f(opcode, a, b, carry)
